# baseline (speedup 1.0000x reference)
import jax
import jax.numpy as jnp
from jax import lax
from jax.experimental import pallas as pl
from jax.experimental.pallas import tpu as pltpu
from jax.experimental.pallas import tpu_sc as plsc

N_NODES = 50000
HIDDEN = 64
NUM_CLASSES = 4
CLIN_DIM = 128
MEL_DIM = 512
N_EDGES = 800000

NC = 2
NS = 16
NPAD = 51200
ROWS_PER_SUB = NPAD // NS
QW = 16
NQ = HIDDEN // QW

CHUNK = 1000
CPAD = 1008
NBLK = N_EDGES // CHUNK
NB = 4

R = 2000
GRID = 25
RP = 1024
RP8 = 256


def _mesh():
    return plsc.VectorSubcoreMesh(core_axis_name="c", subcore_axis_name="s")


def _ring_loop(table_hbm, splane, dplane, acc, sbuf, dbuf, rows,
               srcsem, dstsem, gsem, csem, base_blk, nblk, q):
    csl = pl.ds(0, CHUNK)

    def src_load(b, g):
        pltpu.async_copy(splane.at[g], sbuf.at[b, csl], srcsem.at[b])

    def dst_load(b, g):
        pltpu.async_copy(dplane.at[g], dbuf.at[b], dstsem.at[b])

    def src_wait(b):
        pltpu.make_async_copy(splane.at[base_blk], sbuf.at[b, csl],
                              srcsem.at[b]).wait()

    def dst_wait(b):
        pltpu.make_async_copy(dplane.at[base_blk], dbuf.at[b],
                              dstsem.at[b]).wait()

    def gather(b):
        return pltpu.make_async_copy(table_hbm.at[sbuf.at[b, csl]],
                                     rows.at[b], gsem.at[b])

    def scat(b):
        return pltpu.make_async_copy(rows.at[b], acc.at[dbuf.at[b]],
                                     csem.at[b])

    def transform(b):
        if q is not None:
            for i in range(CPAD // 16):
                sl = pl.ds(i * 16, 16)
                sbuf[b, sl] = sbuf[b, sl] * 4 + q

    for t in range(3):
        src_load(t, base_blk + t)
        dst_load(t, base_blk + t)
    for t in range(2):
        src_wait(t)
        transform(t)
        gather(t).start()

    def step(j, _):
        b = lax.rem(j, NB)
        bg = lax.rem(j + 2, NB)
        bs = lax.rem(j + 3, NB)
        dst_wait(b)
        gather(b).wait()
        pltpu.async_copy(rows.at[b], acc.at[dbuf.at[b]], csem.at[b],
                         add=True)

        @pl.when(jnp.logical_and(j >= 1, j + 3 < nblk))
        def _():
            scat(bs).wait()

        @pl.when(j + 3 < nblk)
        def _():
            src_load(bs, base_blk + j + 3)
            dst_load(bs, base_blk + j + 3)

        @pl.when(j + 2 < nblk)
        def _():
            src_wait(bg)
            transform(bg)
            gather(bg).start()
        return 0

    lax.fori_loop(0, nblk, step, 0)
    for k in range(4):
        scat((nblk - 4 + k) % NB).wait()


def _deg_body(sd_hbm, zero1_hbm, ones_hbm, out_hbm, acc, idx_v, ones_v):
    c = lax.axis_index("c")
    s = lax.axis_index("s")
    r0 = s * ROWS_PER_SUB
    pltpu.sync_copy(zero1_hbm, acc.at[pl.ds(r0, ROWS_PER_SUB)])
    pltpu.sync_copy(ones_hbm, ones_v)
    plsc.subcore_barrier()

    nblk_w = NBLK // (NC * NS)
    sb = 5
    base = (c * NS + s) * nblk_w

    def step(k, _):
        pltpu.sync_copy(sd_hbm.at[1, pl.ds(base + k * sb, sb)], idx_v)
        for i in range(sb):
            pltpu.sync_copy(ones_v, acc.at[idx_v.at[i]], add=True)
        return 0
    lax.fori_loop(0, nblk_w // sb, step, 0)

    plsc.subcore_barrier()
    pltpu.sync_copy(acc.at[pl.ds(r0, ROWS_PER_SUB)],
                    out_hbm.at[c, pl.ds(r0, ROWS_PER_SUB)])


def _degree(sd):
    f = pl.kernel(
        _deg_body,
        out_type=jax.ShapeDtypeStruct((NC, NPAD), jnp.float32),
        mesh=_mesh(),
        scratch_types=[
            pltpu.VMEM_SHARED((NPAD,), jnp.float32),
            pltpu.VMEM((5, CHUNK), jnp.int32),
            pltpu.VMEM((CHUNK,), jnp.float32),
        ],
        compiler_params=pltpu.CompilerParams(use_tc_tiling_on_sc=False),
        name="sc_degree",
    )
    zero1 = jnp.zeros((ROWS_PER_SUB,), jnp.float32)
    ones = jnp.ones((CHUNK,), jnp.float32)
    return f(sd, zero1, ones)


def _mp1_body(sd_hbm, hs4_hbm, zero2_hbm, out_hbm,
              acc, sbuf, dbuf, rows, srcsem, dstsem, gsem, csem):
    c = lax.axis_index("c")
    s = lax.axis_index("s")
    r0 = s * ROWS_PER_SUB
    rows_slice = pl.ds(r0, ROWS_PER_SUB)
    pltpu.sync_copy(zero2_hbm, acc.at[rows_slice])
    plsc.subcore_barrier()

    nblk_s = NBLK // NS
    base = s * nblk_s
    splane = sd_hbm.at[0]
    dplane = sd_hbm.at[1]

    def run_quarter(q, out_plane):
        _ring_loop(hs4_hbm, splane, dplane, acc, sbuf, dbuf, rows,
                   srcsem, dstsem, gsem, csem, base, nblk_s, q)
        plsc.subcore_barrier()
        pltpu.sync_copy(acc.at[rows_slice], out_hbm.at[rows_slice, out_plane])

    def run_core(qa, qb):
        run_quarter(qa, qa)
        pltpu.sync_copy(zero2_hbm, acc.at[rows_slice])
        plsc.subcore_barrier()
        run_quarter(qb, qb)

    @pl.when(c == 0)
    def _():
        run_core(0, 1)

    @pl.when(c == 1)
    def _():
        run_core(2, 3)


def _message_pass1(sd, hs4):
    f = pl.kernel(
        _mp1_body,
        out_type=jax.ShapeDtypeStruct((NPAD, NQ, QW), jnp.float32),
        mesh=_mesh(),
        scratch_types=[
            pltpu.VMEM_SHARED((NPAD, QW), jnp.float32),
            pltpu.VMEM((NB, CPAD), jnp.int32),
            pltpu.VMEM((NB, CHUNK), jnp.int32),
            pltpu.VMEM((NB, CHUNK, QW), jnp.float32),
            pltpu.SemaphoreType.DMA((NB,)),
            pltpu.SemaphoreType.DMA((NB,)),
            pltpu.SemaphoreType.DMA((NB,)),
            pltpu.SemaphoreType.DMA((NB,)),
        ],
        compiler_params=pltpu.CompilerParams(use_tc_tiling_on_sc=False),
        name="sc_message_pass1",
    )
    zero2 = jnp.zeros((ROWS_PER_SUB, QW), jnp.float32)
    return f(sd, hs4, zero2)


def _mp2_body(sd_hbm, hs2_hbm, zero2_hbm, out_hbm,
              acc, sbuf, dbuf, rows, srcsem, dstsem, gsem, csem):
    c = lax.axis_index("c")
    s = lax.axis_index("s")
    r0 = s * ROWS_PER_SUB
    rows_slice = pl.ds(r0, ROWS_PER_SUB)
    pltpu.sync_copy(zero2_hbm, acc.at[rows_slice])
    plsc.subcore_barrier()

    nblk_w = NBLK // (NC * NS)
    base = (c * NS + s) * nblk_w
    _ring_loop(hs2_hbm, sd_hbm.at[0], sd_hbm.at[1], acc, sbuf, dbuf, rows,
               srcsem, dstsem, gsem, csem, base, nblk_w, None)

    plsc.subcore_barrier()
    pltpu.sync_copy(acc.at[rows_slice], out_hbm.at[c, rows_slice])


def _message_pass2(sd, hs2p):
    f = pl.kernel(
        _mp2_body,
        out_type=jax.ShapeDtypeStruct((NC, NPAD, QW), jnp.float32),
        mesh=_mesh(),
        scratch_types=[
            pltpu.VMEM_SHARED((NPAD, QW), jnp.float32),
            pltpu.VMEM((NB, CPAD), jnp.int32),
            pltpu.VMEM((NB, CHUNK), jnp.int32),
            pltpu.VMEM((NB, CHUNK, QW), jnp.float32),
            pltpu.SemaphoreType.DMA((NB,)),
            pltpu.SemaphoreType.DMA((NB,)),
            pltpu.SemaphoreType.DMA((NB,)),
            pltpu.SemaphoreType.DMA((NB,)),
        ],
        compiler_params=pltpu.CompilerParams(use_tc_tiling_on_sc=False),
        name="sc_message_pass2",
    )
    zero2 = jnp.zeros((ROWS_PER_SUB, QW), jnp.float32)
    return f(sd, hs2p, zero2)


def _tc1_body(mel_ref, clin_ref, degp_ref, wm_ref, bm_ref, wcc_ref, wcm_ref,
              bc_ref, w1_ref, hs_ref, dinv_ref):
    m = jnp.maximum(
        jnp.dot(mel_ref[...], wm_ref[...], preferred_element_type=jnp.float32)
        + bm_ref[...], 0.0)
    x = jnp.maximum(
        jnp.dot(clin_ref[...], wcc_ref[...], preferred_element_type=jnp.float32)
        + jnp.dot(m, wcm_ref[...], preferred_element_type=jnp.float32)
        + bc_ref[...], 0.0)
    h1 = jnp.dot(x, w1_ref[...], preferred_element_type=jnp.float32)
    deg = degp_ref[0] + degp_ref[1] + 1.0
    dinv = lax.rsqrt(deg)
    hs_ref[...] = h1 * dinv
    dinv_ref[...] = dinv


def _tc1(mel, clinical, degp3, Wm, bm2, Wcc, Wcm, bc2, W1):
    return pl.pallas_call(
        _tc1_body,
        grid=(GRID,),
        in_specs=[
            pl.BlockSpec((R, MEL_DIM), lambda i: (i, 0)),
            pl.BlockSpec((R, CLIN_DIM), lambda i: (i, 0)),
            pl.BlockSpec((NC, R, 1), lambda i: (0, i, 0)),
            pl.BlockSpec((MEL_DIM, HIDDEN), lambda i: (0, 0)),
            pl.BlockSpec((1, HIDDEN), lambda i: (0, 0)),
            pl.BlockSpec((CLIN_DIM, HIDDEN), lambda i: (0, 0)),
            pl.BlockSpec((HIDDEN, HIDDEN), lambda i: (0, 0)),
            pl.BlockSpec((1, HIDDEN), lambda i: (0, 0)),
            pl.BlockSpec((HIDDEN, HIDDEN), lambda i: (0, 0)),
        ],
        out_specs=[
            pl.BlockSpec((R, HIDDEN), lambda i: (i, 0)),
            pl.BlockSpec((R, 1), lambda i: (i, 0)),
        ],
        out_shape=[
            jax.ShapeDtypeStruct((N_NODES, HIDDEN), jnp.float32),
            jax.ShapeDtypeStruct((N_NODES, 1), jnp.float32),
        ],
        name="tc_fused_mlp",
    )(mel, clinical, degp3, Wm, bm2, Wcc, Wcm, bc2, W1)


def _tc2_body(s1_ref, hs_ref, dr_ref, b1_ref, w2_ref, hs2_ref):
    dr = dr_ref[...]
    x2 = jnp.maximum(dr * (s1_ref[...] + hs_ref[...]) + b1_ref[...], 0.0)
    hs2_ref[...] = jnp.dot(dr * x2, w2_ref[...],
                           preferred_element_type=jnp.float32)


def _tc2(s1pack, hspack, dinvrep64, b1p2, W2bd):
    return pl.pallas_call(
        _tc2_body,
        grid=(GRID,),
        in_specs=[
            pl.BlockSpec((RP, 128), lambda i: (i, 0)),
            pl.BlockSpec((RP, 128), lambda i: (i, 0)),
            pl.BlockSpec((RP, 128), lambda i: (i, 0)),
            pl.BlockSpec((1, 128), lambda i: (0, 0)),
            pl.BlockSpec((128, 2 * QW), lambda i: (0, 0)),
        ],
        out_specs=pl.BlockSpec((RP, 2 * QW), lambda i: (i, 0)),
        out_shape=jax.ShapeDtypeStruct((N_NODES // 2, 2 * QW), jnp.float32),
        name="tc_layer2_prep",
    )(s1pack, hspack, dinvrep64, b1p2, W2bd)


def _tc3_body(s2_ref, hs2_ref, dr_ref, b2_ref, out_ref):
    tot = s2_ref[0] + s2_ref[1] + hs2_ref[...]
    out_ref[...] = dr_ref[...] * tot + b2_ref[...]


def _tc3(s2pack, hs2pack, dinvrep16, b2p8):
    return pl.pallas_call(
        _tc3_body,
        grid=(GRID,),
        in_specs=[
            pl.BlockSpec((NC, RP8, 128), lambda i: (0, i, 0)),
            pl.BlockSpec((RP8, 128), lambda i: (i, 0)),
            pl.BlockSpec((RP8, 128), lambda i: (i, 0)),
            pl.BlockSpec((1, 128), lambda i: (0, 0)),
        ],
        out_specs=pl.BlockSpec((RP8, 128), lambda i: (i, 0)),
        out_shape=jax.ShapeDtypeStruct((N_NODES * QW // 128, 128), jnp.float32),
        name="tc_finalize",
    )(s2pack, hs2pack, dinvrep16, b2p8)


def kernel(clinical, mel, edge_index, Wm, bm, Wc, bc, W1, b1, W2, b2):
    sd = edge_index.astype(jnp.int32).reshape(2, NBLK, CHUNK)

    degp = _degree(sd)
    degp3 = degp[:, :N_NODES].reshape(NC, N_NODES, 1)

    bm2 = bm.reshape(1, HIDDEN)
    bc2 = bc.reshape(1, HIDDEN)
    b1p2 = jnp.tile(b1, 2).reshape(1, 128)
    Wcc = Wc[:CLIN_DIM]
    Wcm = Wc[CLIN_DIM:]
    W2p = jnp.pad(W2, ((0, 0), (0, QW - NUM_CLASSES)))
    W2bd = jax.scipy.linalg.block_diag(W2p, W2p)
    b2p8 = jnp.tile(jnp.pad(b2, (0, QW - NUM_CLASSES)), 8).reshape(1, 128)

    hs, dinv = _tc1(mel, clinical, degp3, Wm, bm2, Wcc, Wcm, bc2, W1)
    hs4 = hs.reshape(NQ * N_NODES, QW)
    dinv1 = dinv.reshape(-1)
    dinvrep64 = jnp.repeat(dinv1, HIDDEN).reshape(N_NODES // 2, 128)
    dinvrep16 = jnp.repeat(dinv1, QW).reshape(N_NODES * QW // 128, 128)

    sq = _message_pass1(sd, hs4)
    s1pack = sq.reshape(NPAD * NQ * QW // 128, 128)[:N_NODES // 2]
    hspack = hs4.reshape(NQ * N_NODES * QW // 128, 128)

    hs2p = _tc2(s1pack, hspack, dinvrep64, b1p2, W2bd)
    hs2lin = hs2p.reshape(N_NODES, QW)

    s2p = _message_pass2(sd, hs2lin)
    s2pack = s2p.reshape(NC, NPAD * QW // 128, 128)[:, :N_NODES * QW // 128]

    outp = _tc3(s2pack, hs2pack=_pack_hs2(hs2lin), dinvrep16=dinvrep16,
                b2p8=b2p8)
    return outp.reshape(N_NODES, QW)[:, :NUM_CLASSES]


def _pack_hs2(hs2lin):
    return hs2lin.reshape(N_NODES * QW // 128, 128)


# speedup vs baseline: 1.3107x; 1.3107x over previous
import jax
import jax.numpy as jnp
from jax import lax
from jax.experimental import pallas as pl
from jax.experimental.pallas import tpu as pltpu
from jax.experimental.pallas import tpu_sc as plsc

N_NODES = 50000
HIDDEN = 64
NUM_CLASSES = 4
CLIN_DIM = 128
MEL_DIM = 512
N_EDGES = 800000

NC = 2
NS = 16
NPAD = 51200
ROWS_PER_SUB = NPAD // NS
QW = 16
NQ = HIDDEN // QW

CHUNK = 1000
CPAD = 1008
NBLK = N_EDGES // CHUNK
NB = 4

R = 2000
GRID = 25
RP = 1024
RP8 = 256


def _mesh():
    return plsc.VectorSubcoreMesh(core_axis_name="c", subcore_axis_name="s")


def _ring_loop(table_hbm, splane, dplane, acc, sbuf, dbuf, rows,
               srcsem, dstsem, gsem, csem, base_blk, nblk, q):
    csl = pl.ds(0, CHUNK)

    def src_load(b, g):
        pltpu.async_copy(splane.at[g], sbuf.at[b, csl], srcsem.at[b])

    def dst_load(b, g):
        pltpu.async_copy(dplane.at[g], dbuf.at[b], dstsem.at[b])

    def src_wait(b):
        pltpu.make_async_copy(splane.at[base_blk], sbuf.at[b, csl],
                              srcsem.at[b]).wait()

    def dst_wait(b):
        pltpu.make_async_copy(dplane.at[base_blk], dbuf.at[b],
                              dstsem.at[b]).wait()

    def gather(b):
        return pltpu.make_async_copy(table_hbm.at[sbuf.at[b, csl]],
                                     rows.at[b], gsem.at[b])

    def scat(b):
        return pltpu.make_async_copy(rows.at[b], acc.at[dbuf.at[b]],
                                     csem.at[b])

    def transform(b):
        if q is not None:
            for i in range(CPAD // 16):
                sl = pl.ds(i * 16, 16)
                sbuf[b, sl] = sbuf[b, sl] * 4 + q

    for t in range(3):
        src_load(t, base_blk + t)
        dst_load(t, base_blk + t)
    for t in range(2):
        src_wait(t)
        transform(t)
        gather(t).start()

    def step(j, _):
        b = lax.rem(j, NB)
        bg = lax.rem(j + 2, NB)
        bs = lax.rem(j + 3, NB)
        dst_wait(b)
        gather(b).wait()
        pltpu.async_copy(rows.at[b], acc.at[dbuf.at[b]], csem.at[b],
                         add=True)

        @pl.when(jnp.logical_and(j >= 1, j + 3 < nblk))
        def _():
            scat(bs).wait()

        @pl.when(j + 3 < nblk)
        def _():
            src_load(bs, base_blk + j + 3)
            dst_load(bs, base_blk + j + 3)

        @pl.when(j + 2 < nblk)
        def _():
            src_wait(bg)
            transform(bg)
            gather(bg).start()
        return 0

    lax.fori_loop(0, nblk, step, 0)
    for k in range(4):
        scat((nblk - 4 + k) % NB).wait()


def _deg_body(sd_hbm, zero1_hbm, ones_hbm, out_hbm, acc, idx_v, ones_v):
    c = lax.axis_index("c")
    s = lax.axis_index("s")
    r0 = s * ROWS_PER_SUB
    pltpu.sync_copy(zero1_hbm, acc.at[pl.ds(r0, ROWS_PER_SUB)])
    pltpu.sync_copy(ones_hbm, ones_v)
    plsc.subcore_barrier()

    nblk_w = NBLK // (NC * NS)
    sb = 5
    base = (c * NS + s) * nblk_w

    def step(k, _):
        pltpu.sync_copy(sd_hbm.at[1, pl.ds(base + k * sb, sb)], idx_v)
        for i in range(sb):
            pltpu.sync_copy(ones_v, acc.at[idx_v.at[i]], add=True)
        return 0
    lax.fori_loop(0, nblk_w // sb, step, 0)

    plsc.subcore_barrier()
    pltpu.sync_copy(acc.at[pl.ds(r0, ROWS_PER_SUB)],
                    out_hbm.at[c, pl.ds(r0, ROWS_PER_SUB)])


def _degree(sd):
    f = pl.kernel(
        _deg_body,
        out_type=jax.ShapeDtypeStruct((NC, NPAD), jnp.float32),
        mesh=_mesh(),
        scratch_types=[
            pltpu.VMEM_SHARED((NPAD,), jnp.float32),
            pltpu.VMEM((5, CHUNK), jnp.int32),
            pltpu.VMEM((CHUNK,), jnp.float32),
        ],
        compiler_params=pltpu.CompilerParams(use_tc_tiling_on_sc=False),
        name="sc_degree",
    )
    zero1 = jnp.zeros((ROWS_PER_SUB,), jnp.float32)
    ones = jnp.ones((CHUNK,), jnp.float32)
    return f(sd, zero1, ones)


def _mp1_body(sd_hbm, hs4_hbm, zero2_hbm, out_hbm,
              acc, sbuf, dbuf, rows, srcsem, dstsem, gsem, csem):
    c = lax.axis_index("c")
    s = lax.axis_index("s")
    r0 = s * ROWS_PER_SUB
    rows_slice = pl.ds(r0, ROWS_PER_SUB)
    pltpu.sync_copy(zero2_hbm, acc.at[rows_slice])
    plsc.subcore_barrier()

    nblk_s = NBLK // NS
    base = s * nblk_s
    splane = sd_hbm.at[0]
    dplane = sd_hbm.at[1]

    def run_quarter(q, out_plane):
        _ring_loop(hs4_hbm, splane, dplane, acc, sbuf, dbuf, rows,
                   srcsem, dstsem, gsem, csem, base, nblk_s, q)
        plsc.subcore_barrier()
        pltpu.sync_copy(acc.at[rows_slice], out_hbm.at[out_plane, rows_slice])

    def run_core(qa, qb):
        run_quarter(qa, qa)
        pltpu.sync_copy(zero2_hbm, acc.at[rows_slice])
        plsc.subcore_barrier()
        run_quarter(qb, qb)

    @pl.when(c == 0)
    def _():
        run_core(0, 1)

    @pl.when(c == 1)
    def _():
        run_core(2, 3)


def _message_pass1(sd, hs4):
    f = pl.kernel(
        _mp1_body,
        out_type=jax.ShapeDtypeStruct((NQ, NPAD, QW), jnp.float32),
        mesh=_mesh(),
        scratch_types=[
            pltpu.VMEM_SHARED((NPAD, QW), jnp.float32),
            pltpu.VMEM((NB, CPAD), jnp.int32),
            pltpu.VMEM((NB, CHUNK), jnp.int32),
            pltpu.VMEM((NB, CHUNK, QW), jnp.float32),
            pltpu.SemaphoreType.DMA((NB,)),
            pltpu.SemaphoreType.DMA((NB,)),
            pltpu.SemaphoreType.DMA((NB,)),
            pltpu.SemaphoreType.DMA((NB,)),
        ],
        compiler_params=pltpu.CompilerParams(use_tc_tiling_on_sc=False),
        name="sc_message_pass1",
    )
    zero2 = jnp.zeros((ROWS_PER_SUB, QW), jnp.float32)
    return f(sd, hs4, zero2)


def _mp2_body(sd_hbm, hs2_hbm, zero2_hbm, out_hbm,
              acc, sbuf, dbuf, rows, srcsem, dstsem, gsem, csem):
    c = lax.axis_index("c")
    s = lax.axis_index("s")
    r0 = s * ROWS_PER_SUB
    rows_slice = pl.ds(r0, ROWS_PER_SUB)
    pltpu.sync_copy(zero2_hbm, acc.at[rows_slice])
    plsc.subcore_barrier()

    nblk_w = NBLK // (NC * NS)
    base = (c * NS + s) * nblk_w
    _ring_loop(hs2_hbm, sd_hbm.at[0], sd_hbm.at[1], acc, sbuf, dbuf, rows,
               srcsem, dstsem, gsem, csem, base, nblk_w, None)

    plsc.subcore_barrier()
    pltpu.sync_copy(acc.at[rows_slice], out_hbm.at[c, rows_slice])


def _message_pass2(sd, hs2p):
    f = pl.kernel(
        _mp2_body,
        out_type=jax.ShapeDtypeStruct((NC, NPAD, QW), jnp.float32),
        mesh=_mesh(),
        scratch_types=[
            pltpu.VMEM_SHARED((NPAD, QW), jnp.float32),
            pltpu.VMEM((NB, CPAD), jnp.int32),
            pltpu.VMEM((NB, CHUNK), jnp.int32),
            pltpu.VMEM((NB, CHUNK, QW), jnp.float32),
            pltpu.SemaphoreType.DMA((NB,)),
            pltpu.SemaphoreType.DMA((NB,)),
            pltpu.SemaphoreType.DMA((NB,)),
            pltpu.SemaphoreType.DMA((NB,)),
        ],
        compiler_params=pltpu.CompilerParams(use_tc_tiling_on_sc=False),
        name="sc_message_pass2",
    )
    zero2 = jnp.zeros((ROWS_PER_SUB, QW), jnp.float32)
    return f(sd, hs2p, zero2)


def _tc1_body(mel_ref, clin_ref, degp_ref, wm_ref, bm_ref, wcc_ref, wcm_ref,
              bc_ref, w1_ref, hs_ref, dinv_ref):
    m = jnp.maximum(
        jnp.dot(mel_ref[...], wm_ref[...], preferred_element_type=jnp.float32)
        + bm_ref[...], 0.0)
    x = jnp.maximum(
        jnp.dot(clin_ref[...], wcc_ref[...], preferred_element_type=jnp.float32)
        + jnp.dot(m, wcm_ref[...], preferred_element_type=jnp.float32)
        + bc_ref[...], 0.0)
    h1 = jnp.dot(x, w1_ref[...], preferred_element_type=jnp.float32)
    deg = degp_ref[0] + degp_ref[1] + 1.0
    dinv = lax.rsqrt(deg)
    hs_ref[...] = h1 * dinv
    dinv_ref[...] = dinv


def _tc1(mel, clinical, degp3, Wm, bm2, Wcc, Wcm, bc2, W1):
    return pl.pallas_call(
        _tc1_body,
        grid=(GRID,),
        in_specs=[
            pl.BlockSpec((R, MEL_DIM), lambda i: (i, 0)),
            pl.BlockSpec((R, CLIN_DIM), lambda i: (i, 0)),
            pl.BlockSpec((NC, R, 1), lambda i: (0, i, 0)),
            pl.BlockSpec((MEL_DIM, HIDDEN), lambda i: (0, 0)),
            pl.BlockSpec((1, HIDDEN), lambda i: (0, 0)),
            pl.BlockSpec((CLIN_DIM, HIDDEN), lambda i: (0, 0)),
            pl.BlockSpec((HIDDEN, HIDDEN), lambda i: (0, 0)),
            pl.BlockSpec((1, HIDDEN), lambda i: (0, 0)),
            pl.BlockSpec((HIDDEN, HIDDEN), lambda i: (0, 0)),
        ],
        out_specs=[
            pl.BlockSpec((R, HIDDEN), lambda i: (i, 0)),
            pl.BlockSpec((R, 1), lambda i: (i, 0)),
        ],
        out_shape=[
            jax.ShapeDtypeStruct((N_NODES, HIDDEN), jnp.float32),
            jax.ShapeDtypeStruct((N_NODES, 1), jnp.float32),
        ],
        name="tc_fused_mlp",
    )(mel, clinical, degp3, Wm, bm2, Wcc, Wcm, bc2, W1)


def _tc2_body(sq_ref, hs_ref, dinv_ref, b1_ref, w2_ref, hs2_ref):
    s1 = jnp.concatenate(
        [sq_ref[0], sq_ref[1], sq_ref[2], sq_ref[3]], axis=1)
    dinv = dinv_ref[...]
    x2 = jnp.maximum(dinv * (s1 + hs_ref[...]) + b1_ref[...], 0.0)
    h2p = jnp.dot(x2, w2_ref[...], preferred_element_type=jnp.float32)
    hs2_ref[...] = h2p * dinv


def _tc2(sq, hs, dinv, b12, W2p):
    return pl.pallas_call(
        _tc2_body,
        grid=(GRID,),
        in_specs=[
            pl.BlockSpec((NQ, R, QW), lambda i: (0, i, 0)),
            pl.BlockSpec((R, HIDDEN), lambda i: (i, 0)),
            pl.BlockSpec((R, 1), lambda i: (i, 0)),
            pl.BlockSpec((1, HIDDEN), lambda i: (0, 0)),
            pl.BlockSpec((HIDDEN, QW), lambda i: (0, 0)),
        ],
        out_specs=pl.BlockSpec((R, QW), lambda i: (i, 0)),
        out_shape=jax.ShapeDtypeStruct((N_NODES, QW), jnp.float32),
        name="tc_layer2_prep",
    )(sq, hs, dinv, b12, W2p)


def _tc3_body(s2_ref, hs2_ref, dr_ref, b2_ref, out_ref):
    tot = s2_ref[0] + s2_ref[1] + hs2_ref[...]
    out_ref[...] = dr_ref[...] * tot + b2_ref[...]


def _tc3(s2pack, hs2pack, dinvrep16, b2p8):
    return pl.pallas_call(
        _tc3_body,
        grid=(GRID,),
        in_specs=[
            pl.BlockSpec((NC, RP8, 128), lambda i: (0, i, 0)),
            pl.BlockSpec((RP8, 128), lambda i: (i, 0)),
            pl.BlockSpec((RP8, 128), lambda i: (i, 0)),
            pl.BlockSpec((1, 128), lambda i: (0, 0)),
        ],
        out_specs=pl.BlockSpec((RP8, 128), lambda i: (i, 0)),
        out_shape=jax.ShapeDtypeStruct((N_NODES * QW // 128, 128), jnp.float32),
        name="tc_finalize",
    )(s2pack, hs2pack, dinvrep16, b2p8)


def kernel(clinical, mel, edge_index, Wm, bm, Wc, bc, W1, b1, W2, b2):
    sd = edge_index.astype(jnp.int32).reshape(2, NBLK, CHUNK)

    degp = _degree(sd)
    degp3 = degp[:, :N_NODES].reshape(NC, N_NODES, 1)

    bm2 = bm.reshape(1, HIDDEN)
    bc2 = bc.reshape(1, HIDDEN)
    b12 = b1.reshape(1, HIDDEN)
    Wcc = Wc[:CLIN_DIM]
    Wcm = Wc[CLIN_DIM:]
    W2p = jnp.pad(W2, ((0, 0), (0, QW - NUM_CLASSES)))
    b2p8 = jnp.tile(jnp.pad(b2, (0, QW - NUM_CLASSES)), 8).reshape(1, 128)

    hs, dinv = _tc1(mel, clinical, degp3, Wm, bm2, Wcc, Wcm, bc2, W1)
    hs4 = hs.reshape(NQ * N_NODES, QW)
    dinvrep16 = jnp.repeat(dinv.reshape(-1), QW).reshape(
        N_NODES * QW // 128, 128)

    sq = _message_pass1(sd, hs4)

    hs2p = _tc2(sq, hs, dinv, b12, W2p)

    s2p = _message_pass2(sd, hs2p)
    s2pack = s2p.reshape(NC, NPAD * QW // 128, 128)
    hs2pack = hs2p.reshape(N_NODES * QW // 128, 128)

    outp = _tc3(s2pack, hs2pack, dinvrep16, b2p8)
    return outp.reshape(N_NODES, QW)[:, :NUM_CLASSES]
